# baseline (speedup 1.0000x reference)
import functools

import jax
import jax.numpy as jnp
from jax import lax
from jax.experimental import pallas as pl
from jax.experimental.pallas import tpu as pltpu
from jax.experimental.pallas import tpu_sc as plsc

N_NODES = 10000
D_FEAT = 256
N_EDGES = 160000

NC = 2
NS = 16
NW = NC * NS
L = 16

C = 64
_CHUNKS_PER_SET = -(-N_EDGES // (NW * C))
EPW = _CHUNKS_PER_SET * C
EPAD = EPW * NW
M = 2 * EPAD
EPW2 = 2 * EPW
K2 = 2 * _CHUNKS_PER_SET


def _sc_body(z_hbm, eidx_hbm, out_hbm,
             zsp, eidx, idxbuf, rows, obuf, semg0, semg1, semo0, semo1):
    wid = lax.axis_index("s") * NC + lax.axis_index("c")
    base = wid * EPW2

    pltpu.sync_copy(eidx_hbm.at[pl.ds(base, EPW2)], eidx)

    @pl.when(lax.axis_index("s") == 0)
    def _():
        pltpu.sync_copy(z_hbm, zsp)

    plsc.subcore_barrier()

    semg = (semg0, semg1)
    semo = (semo0, semo1)

    def unpack_idx(k, b):
        for w in range(C // L):
            v = eidx[pl.ds(k * C + w * L, L)]
            idxbuf[2 * b, pl.ds(w * L, L)] = v & 0xFFFF
            idxbuf[2 * b + 1, pl.ds(w * L, L)] = lax.shift_right_logical(v, 16)

    def issue(k, b):
        unpack_idx(k, b)
        pltpu.make_async_copy(
            zsp.at[idxbuf.at[2 * b]], rows.at[2 * b], semg[b]).start()
        pltpu.make_async_copy(
            zsp.at[idxbuf.at[2 * b + 1]], rows.at[2 * b + 1], semg[b]).start()

    def wait_gather(b):
        pltpu.make_async_copy(
            zsp.at[idxbuf.at[2 * b]], rows.at[2 * b], semg[b]).wait()
        pltpu.make_async_copy(
            zsp.at[idxbuf.at[2 * b + 1]], rows.at[2 * b + 1], semg[b]).wait()

    def compute(k, b):
        lane = lax.iota(jnp.int32, L)

        @plsc.parallel_loop(0, C // L, unroll=4)
        def group(q):
            e0 = q * L
            vals = jnp.zeros((L,), jnp.float32)
            for r in range(L):
                e = e0 + r
                accs = [None] * 4
                for g in range(8):
                    vs = plsc.bitcast(
                        rows[2 * b, e, pl.ds(g * L, L)], jnp.bfloat16)
                    vd = plsc.bitcast(
                        rows[2 * b + 1, e, pl.ds(g * L, L)], jnp.bfloat16)
                    p_lo, p_hi = plsc.unpack(
                        vs * vd, format=plsc.PackFormat.INTERLEAVED)
                    j0 = 2 * (g % 2)
                    accs[j0] = p_lo if accs[j0] is None else accs[j0] + p_lo
                    accs[j0 + 1] = (
                        p_hi if accs[j0 + 1] is None else accs[j0 + 1] + p_hi)
                tot = (accs[0] + accs[1]) + (accs[2] + accs[3])
                vals = jnp.where(lane == r, jnp.sum(tot), vals)
            obuf[b, pl.ds(e0, L)] = vals

    def start_out(k, b):
        pltpu.make_async_copy(
            obuf.at[b], out_hbm.at[pl.ds(base + k * C, C)], semo[b]).start()

    def wait_out(b):
        pltpu.make_async_copy(
            obuf.at[b], out_hbm.at[pl.ds(base, C)], semo[b]).wait()

    issue(0, 0)
    issue(1, 1)

    def outer(i, carry):
        g = i * 2
        for b in range(2):
            k = g + b
            wait_gather(b)

            @pl.when(k >= 2)
            def _():
                wait_out(b)

            compute(k, b)
            start_out(k, b)

            @pl.when(k + 2 < K2)
            def _():
                issue(k + 2, b)
        return carry

    lax.fori_loop(0, K2 // 2, outer, 0)

    wait_out(0)
    wait_out(1)


_sc_gather_dot = functools.partial(
    pl.kernel,
    out_type=jax.ShapeDtypeStruct((M,), jnp.float32),
    mesh=plsc.VectorSubcoreMesh(core_axis_name="c", subcore_axis_name="s"),
    compiler_params=pltpu.CompilerParams(needs_layout_passes=False),
    scratch_types=[
        pltpu.VMEM_SHARED((N_NODES, D_FEAT // 2), jnp.int32),
        pltpu.VMEM((EPW2,), jnp.int32),
        pltpu.VMEM((4, C), jnp.int32),
        pltpu.VMEM((4, C, D_FEAT // 2), jnp.int32),
        pltpu.VMEM((2, C), jnp.float32),
        pltpu.SemaphoreType.DMA,
        pltpu.SemaphoreType.DMA,
        pltpu.SemaphoreType.DMA,
        pltpu.SemaphoreType.DMA,
    ],
)(_sc_body)


def _loss_body(pos_ref, neg_ref, out_ref):
    eps = 1e-15
    x = pos_ref[...]
    s = 1.0 / (1.0 + jnp.exp(-x))
    pos_loss = -jnp.sum(jnp.log(s + eps)) / N_EDGES
    y = neg_ref[...]
    t = 1.0 / (1.0 + jnp.exp(-y))
    neg_loss = -jnp.sum(jnp.log(1.0 - t + eps)) / N_EDGES
    out_ref[0, 0] = pos_loss + neg_loss


_loss_reduce = pl.pallas_call(
    _loss_body,
    out_shape=jax.ShapeDtypeStruct((1, 1), jnp.float32),
    out_specs=pl.BlockSpec(memory_space=pltpu.SMEM),
)


def kernel(z, pos_edge_index, neg_edge_index):
    z16 = z.astype(jnp.bfloat16)
    zpacked = lax.bitcast_convert_type(
        z16.reshape(N_NODES, D_FEAT // 2, 2), jnp.int32)
    pad = jnp.zeros((EPAD - N_EDGES,), jnp.int32)
    pos_packed = pos_edge_index[0] | (pos_edge_index[1] << 16)
    neg_packed = neg_edge_index[0] | (neg_edge_index[1] << 16)
    eidx = jnp.concatenate([pos_packed, pad, neg_packed, pad])
    logits = _sc_gather_dot(zpacked, eidx)
    pos_logits = logits[:N_EDGES].reshape(1250, 128)
    neg_logits = logits[EPAD:EPAD + N_EDGES].reshape(1250, 128)
    loss = _loss_reduce(pos_logits, neg_logits)
    return loss[0, 0]


# speedup vs baseline: 1.5719x; 1.5719x over previous
import functools

import jax
import jax.numpy as jnp
from jax import lax
from jax.experimental import pallas as pl
from jax.experimental.pallas import tpu as pltpu
from jax.experimental.pallas import tpu_sc as plsc

N_NODES = 10000
D_FEAT = 256
N_EDGES = 160000

NC = 2
NS = 16
NW = NC * NS
L = 16

C = 64
_CHUNKS_PER_SET = -(-N_EDGES // (NW * C))
EPW = _CHUNKS_PER_SET * C
EPAD = EPW * NW
M = 2 * EPAD
EPW2 = 2 * EPW
K2 = 2 * _CHUNKS_PER_SET


def _sc_body(z_hbm, eidx_hbm, out_hbm,
             zsp, eidx, si0, si1, di0, di1, o0, o1,
             r0, r1, r2, r3, semg0, semg1, semo0, semo1):
    wid = lax.axis_index("s") * NC + lax.axis_index("c")
    base = wid * EPW2

    pltpu.sync_copy(eidx_hbm.at[pl.ds(base, EPW2)], eidx)

    @pl.when(lax.axis_index("s") == 0)
    def _():
        pltpu.sync_copy(z_hbm, zsp)

    plsc.subcore_barrier()

    bufs = (
        (si0, di0, o0, (r0, r1), semg0, semo0),
        (si1, di1, o1, (r2, r3), semg1, semo1),
    )

    def unpack_idx(k, b):
        sib, dib = bufs[b][0], bufs[b][1]
        for w in range(C // L):
            v = eidx[pl.ds(k * C + w * L, L)]
            sib[pl.ds(w * L, L)] = v & 0xFFFF
            dib[pl.ds(w * L, L)] = lax.shift_right_logical(v, 16)

    def issue(k, b):
        sib, dib, _, (rs, rd), semg, _ = bufs[b]
        unpack_idx(k, b)
        pltpu.make_async_copy(zsp.at[sib], rs, semg).start()
        pltpu.make_async_copy(zsp.at[dib], rd, semg).start()

    def wait_gather(b):
        sib, dib, _, (rs, rd), semg, _ = bufs[b]
        pltpu.make_async_copy(zsp.at[sib], rs, semg).wait()
        pltpu.make_async_copy(zsp.at[dib], rd, semg).wait()

    def compute(k, b):
        _, _, ob, (rs, rd), _, _ = bufs[b]
        lane = lax.iota(jnp.int32, L)

        @plsc.parallel_loop(0, C // L, unroll=2)
        def group(q):
            e0 = q * L
            vals = jnp.zeros((L,), jnp.float32)
            for r in range(L):
                e = e0 + r
                accs = [None] * 4
                for g in range(8):
                    vs = plsc.bitcast(rs[e, pl.ds(g * L, L)], jnp.bfloat16)
                    vd = plsc.bitcast(rd[e, pl.ds(g * L, L)], jnp.bfloat16)
                    p_lo, p_hi = plsc.unpack(
                        vs * vd, format=plsc.PackFormat.INTERLEAVED)
                    j0 = 2 * (g % 2)
                    accs[j0] = p_lo if accs[j0] is None else accs[j0] + p_lo
                    accs[j0 + 1] = (
                        p_hi if accs[j0 + 1] is None else accs[j0 + 1] + p_hi)
                tot = (accs[0] + accs[1]) + (accs[2] + accs[3])
                vals = jnp.where(lane == r, jnp.sum(tot), vals)
            ob[pl.ds(e0, L)] = vals

    def start_out(k, b):
        ob, semo = bufs[b][2], bufs[b][5]
        pltpu.make_async_copy(
            ob, out_hbm.at[pl.ds(base + k * C, C)], semo).start()

    def wait_out(b):
        ob, semo = bufs[b][2], bufs[b][5]
        pltpu.make_async_copy(
            ob, out_hbm.at[pl.ds(base, C)], semo).wait()

    issue(0, 0)
    issue(1, 1)

    def outer(i, carry):
        g = i * 2
        for b in range(2):
            k = g + b
            wait_gather(b)

            @pl.when(k >= 2)
            def _():
                wait_out(b)

            compute(k, b)
            start_out(k, b)

            @pl.when(k + 2 < K2)
            def _():
                issue(k + 2, b)
        return carry

    lax.fori_loop(0, K2 // 2, outer, 0)

    wait_out(0)
    wait_out(1)


_sc_gather_dot = functools.partial(
    pl.kernel,
    out_type=jax.ShapeDtypeStruct((M,), jnp.float32),
    mesh=plsc.VectorSubcoreMesh(core_axis_name="c", subcore_axis_name="s"),
    compiler_params=pltpu.CompilerParams(needs_layout_passes=False),
    scratch_types=[
        pltpu.VMEM_SHARED((N_NODES, D_FEAT // 2), jnp.int32),
        pltpu.VMEM((EPW2,), jnp.int32),
        pltpu.VMEM((C,), jnp.int32),
        pltpu.VMEM((C,), jnp.int32),
        pltpu.VMEM((C,), jnp.int32),
        pltpu.VMEM((C,), jnp.int32),
        pltpu.VMEM((C,), jnp.float32),
        pltpu.VMEM((C,), jnp.float32),
        pltpu.VMEM((C, D_FEAT // 2), jnp.int32),
        pltpu.VMEM((C, D_FEAT // 2), jnp.int32),
        pltpu.VMEM((C, D_FEAT // 2), jnp.int32),
        pltpu.VMEM((C, D_FEAT // 2), jnp.int32),
        pltpu.SemaphoreType.DMA,
        pltpu.SemaphoreType.DMA,
        pltpu.SemaphoreType.DMA,
        pltpu.SemaphoreType.DMA,
    ],
)(_sc_body)


def _loss_body(pos_ref, neg_ref, out_ref):
    eps = 1e-15
    x = pos_ref[...]
    s = 1.0 / (1.0 + jnp.exp(-x))
    pos_loss = -jnp.sum(jnp.log(s + eps)) / N_EDGES
    y = neg_ref[...]
    t = 1.0 / (1.0 + jnp.exp(-y))
    neg_loss = -jnp.sum(jnp.log(1.0 - t + eps)) / N_EDGES
    out_ref[0, 0] = pos_loss + neg_loss


_loss_reduce = pl.pallas_call(
    _loss_body,
    out_shape=jax.ShapeDtypeStruct((1, 1), jnp.float32),
    out_specs=pl.BlockSpec(memory_space=pltpu.SMEM),
)


def kernel(z, pos_edge_index, neg_edge_index):
    z16 = z.astype(jnp.bfloat16)
    zpacked = lax.bitcast_convert_type(
        z16.reshape(N_NODES, D_FEAT // 2, 2), jnp.int32)
    pad = jnp.zeros((EPAD - N_EDGES,), jnp.int32)
    pos_packed = pos_edge_index[0] | (pos_edge_index[1] << 16)
    neg_packed = neg_edge_index[0] | (neg_edge_index[1] << 16)
    eidx = jnp.concatenate([pos_packed, pad, neg_packed, pad])
    logits = _sc_gather_dot(zpacked, eidx)
    pos_logits = logits[:N_EDGES].reshape(1250, 128)
    neg_logits = logits[EPAD:EPAD + N_EDGES].reshape(1250, 128)
    loss = _loss_reduce(pos_logits, neg_logits)
    return loss[0, 0]
